# baseline (speedup 1.0000x reference)
import functools

import jax
import jax.numpy as jnp
from jax import lax
from jax.experimental import pallas as pl
from jax.experimental.pallas import tpu as pltpu
from jax.experimental.pallas import tpu_sc as plsc

NUM_CORES = 2
NUM_SUBCORES = 16
LANES = 16
CHUNK = 32768


def kernel(x, embed_table):
    batch, seq_len, d_model = x.shape
    n_tbl = embed_table.shape[0]
    n_workers = NUM_CORES * NUM_SUBCORES
    rows_per_w = seq_len // n_workers
    row_elems = seq_len * d_model
    n_chunks = row_elems // CHUNK

    x_flat = x.reshape(row_elems)
    tbl_flat = embed_table.reshape(n_tbl * d_model)

    mesh = plsc.VectorSubcoreMesh(
        core_axis_name="c", subcore_axis_name="s"
    )

    @functools.partial(
        pl.kernel,
        mesh=mesh,
        out_type=jax.ShapeDtypeStruct((seq_len * row_elems,), jnp.float32),
        scratch_types=[
            pltpu.VMEM((CHUNK,), jnp.float32),
            pltpu.VMEM((CHUNK,), jnp.float32),
        ],
    )
    def rpe_sc(x_hbm, tbl_hbm, out_hbm, xb, tb):
        wid = lax.axis_index("s") * NUM_CORES + lax.axis_index("c")
        i0 = wid * rows_per_w

        def chunk_body(c, _):
            j_off = c * CHUNK
            pltpu.sync_copy(x_hbm.at[pl.ds(j_off, CHUNK)], xb)

            def row_body(r, _):
                i = i0 + r
                t_off = (seq_len - i) * d_model + j_off
                pltpu.sync_copy(tbl_hbm.at[pl.ds(t_off, CHUNK)], tb)

                def add_body(v, _):
                    o = v * LANES
                    tb[pl.ds(o, LANES)] = (
                        tb[pl.ds(o, LANES)] + xb[pl.ds(o, LANES)]
                    )
                    return 0

                lax.fori_loop(0, CHUNK // LANES, add_body, 0)
                pltpu.sync_copy(
                    tb, out_hbm.at[pl.ds(i * row_elems + j_off, CHUNK)]
                )
                return 0

            lax.fori_loop(0, rows_per_w, row_body, 0)
            return 0

        lax.fori_loop(0, n_chunks, chunk_body, 0)

    out_flat = rpe_sc(x_flat, tbl_flat)
    return out_flat.reshape(seq_len, seq_len, d_model)


# speedup vs baseline: 1.8468x; 1.8468x over previous
import functools

import jax
import jax.numpy as jnp
from jax import lax
from jax.experimental import pallas as pl
from jax.experimental.pallas import tpu as pltpu
from jax.experimental.pallas import tpu_sc as plsc

NUM_CORES = 2
NUM_SUBCORES = 16
LANES = 16
CHUNK = 16384
NBUF = 4


def kernel(x, embed_table):
    batch, seq_len, d_model = x.shape
    n_tbl = embed_table.shape[0]
    n_workers = NUM_CORES * NUM_SUBCORES
    rows_per_w = seq_len // n_workers
    row_elems = seq_len * d_model
    n_chunks = row_elems // CHUNK

    x_flat = x.reshape(row_elems)
    tbl_flat = embed_table.reshape(n_tbl * d_model)

    mesh = plsc.VectorSubcoreMesh(
        core_axis_name="c", subcore_axis_name="s"
    )

    @functools.partial(
        pl.kernel,
        mesh=mesh,
        out_type=jax.ShapeDtypeStruct((seq_len * row_elems,), jnp.float32),
        scratch_types=[
            pltpu.VMEM((CHUNK,), jnp.float32),
            [pltpu.VMEM((CHUNK,), jnp.float32) for _ in range(NBUF)],
            [pltpu.SemaphoreType.DMA for _ in range(NBUF)],
            [pltpu.SemaphoreType.DMA for _ in range(NBUF)],
        ],
    )
    def rpe_sc(x_hbm, tbl_hbm, out_hbm, xb, tbufs, sems_in, sems_out):
        wid = lax.axis_index("s") * NUM_CORES + lax.axis_index("c")
        i0 = wid * rows_per_w

        def chunk_body(c, _):
            j_off = c * CHUNK
            pltpu.sync_copy(x_hbm.at[pl.ds(j_off, CHUNK)], xb)

            def in_copy(r):
                t_off = (seq_len - (i0 + r)) * d_model + j_off
                return pltpu.make_async_copy(
                    tbl_hbm.at[pl.ds(t_off, CHUNK)],
                    tbufs[r % NBUF],
                    sems_in[r % NBUF],
                )

            def out_copy(r):
                o_off = (i0 + r) * row_elems + j_off
                return pltpu.make_async_copy(
                    tbufs[r % NBUF],
                    out_hbm.at[pl.ds(o_off, CHUNK)],
                    sems_out[r % NBUF],
                )

            in_copy(0).start()
            in_copy(1).start()
            for r in range(rows_per_w):
                b = r % NBUF
                in_copy(r).wait()

                def add_body(v, _):
                    o = v * LANES
                    plsc.addupdate(
                        tbufs[b].at[pl.ds(o, LANES)],
                        xb[pl.ds(o, LANES)],
                    )
                    return 0

                lax.fori_loop(
                    0, CHUNK // LANES, add_body, 0, unroll=8
                )
                out_copy(r).start()
                n = r + 2
                if n < rows_per_w:
                    if n >= NBUF:
                        out_copy(n - NBUF).wait()
                    in_copy(n).start()
            for r in range(rows_per_w - NBUF, rows_per_w):
                out_copy(r).wait()
            return 0

        lax.fori_loop(0, n_chunks, chunk_body, 0)

    out_flat = rpe_sc(x_flat, tbl_flat)
    return out_flat.reshape(seq_len, seq_len, d_model)
